# baseline (speedup 1.0000x reference)
import functools

import jax
import jax.numpy as jnp
from jax import lax
from jax.experimental import pallas as pl
from jax.experimental.pallas import tpu as pltpu
from jax.experimental.pallas import tpu_sc as plsc

B = 16384
D = 64
NC = 2
NS = 16
NW = NC * NS
BPW = B // NW
CH = 128
NCH = BPW // CH

_mesh = plsc.VectorSubcoreMesh(core_axis_name="c", subcore_axis_name="s")


@functools.partial(
    pl.kernel,
    mesh=_mesh,
    out_type=(
        jax.ShapeDtypeStruct((B, D), jnp.float32),
        jax.ShapeDtypeStruct((B, D), jnp.float32),
    ),
    scratch_types=[
        pltpu.VMEM((BPW,), jnp.int32),
        pltpu.VMEM((BPW,), jnp.int32),
        pltpu.VMEM((BPW, D), jnp.float32),
        pltpu.VMEM((BPW, D), jnp.float32),
        pltpu.SemaphoreType.DMA,
        pltpu.SemaphoreType.DMA,
    ],
    compiler_params=pltpu.CompilerParams(use_tc_tiling_on_sc=False),
)
def _dual_gather(w_idx_hbm, c_idx_hbm, wt_hbm, ct_hbm, w_out, c_out,
                 widx_v, cidx_v, wrows_v, crows_v, sem_w, sem_c):
    wid = lax.axis_index("s") * NC + lax.axis_index("c")
    base = wid * BPW
    pltpu.sync_copy(w_idx_hbm.at[pl.ds(base, BPW)], widx_v)
    pltpu.sync_copy(c_idx_hbm.at[pl.ds(base, BPW)], cidx_v)
    copies = []
    for j in range(NCH):
        sl = pl.ds(j * CH, CH)
        copies.append(
            pltpu.async_copy(wt_hbm.at[widx_v.at[sl]], wrows_v.at[sl], sem_w))
        copies.append(
            pltpu.async_copy(ct_hbm.at[cidx_v.at[sl]], crows_v.at[sl], sem_c))
    for cp in copies:
        cp.wait()
    pltpu.sync_copy(wrows_v, w_out.at[pl.ds(base, BPW)])
    pltpu.sync_copy(crows_v, c_out.at[pl.ds(base, BPW)])


def kernel(X, word_table, context_table):
    w = X[:, 0]
    c = X[:, 1]
    w_rows, c_rows = _dual_gather(w, c, word_table, context_table)
    return (w_rows[:, None, :], c_rows[:, None, :])


# speedup vs baseline: 1.5775x; 1.5775x over previous
import functools

import jax
import jax.numpy as jnp
from jax import lax
from jax.experimental import pallas as pl
from jax.experimental.pallas import tpu as pltpu
from jax.experimental.pallas import tpu_sc as plsc

B = 16384
D = 64
NC = 2
NS = 16
NW = NC * NS
BPW = B // NW
L = 16
CHR = 256
NCHK = BPW // CHR
NG = CHR // L

_mesh = plsc.VectorSubcoreMesh(core_axis_name="c", subcore_axis_name="s")


@functools.partial(
    pl.kernel,
    mesh=_mesh,
    out_type=(
        jax.ShapeDtypeStruct((B, D), jnp.float32),
        jax.ShapeDtypeStruct((B, D), jnp.float32),
    ),
    scratch_types=[
        pltpu.VMEM((BPW,), jnp.int32),
        pltpu.VMEM((BPW,), jnp.int32),
        pltpu.VMEM((CHR, D), jnp.float32),
        pltpu.VMEM((CHR, D), jnp.float32),
        pltpu.SemaphoreType.DMA,
        pltpu.SemaphoreType.DMA,
    ],
)
def _dual_gather(w_idx_hbm, c_idx_hbm, wt_hbm, ct_hbm, w_out, c_out,
                 widx_v, cidx_v, wrows_v, crows_v, sem_w, sem_c):
    wid = lax.axis_index("s") * NC + lax.axis_index("c")
    base = wid * BPW
    pltpu.sync_copy(w_idx_hbm.at[pl.ds(base, BPW)], widx_v)
    pltpu.sync_copy(c_idx_hbm.at[pl.ds(base, BPW)], cidx_v)

    def chunk(k, _):
        def fire(g, _):
            vw = widx_v[pl.ds(k * CHR + g * L, L)]
            vc = cidx_v[pl.ds(k * CHR + g * L, L)]
            for l in range(L):
                pltpu.async_copy(
                    wt_hbm.at[pl.ds(vw[l], 1)],
                    wrows_v.at[pl.ds(g * L + l, 1)], sem_w)
                pltpu.async_copy(
                    ct_hbm.at[pl.ds(vc[l], 1)],
                    crows_v.at[pl.ds(g * L + l, 1)], sem_c)
            return 0

        lax.fori_loop(0, NG, fire, 0)

        def drain(j, _):
            pltpu.make_async_copy(
                wt_hbm.at[pl.ds(0, 1)], wrows_v.at[pl.ds(0, 1)], sem_w).wait()
            pltpu.make_async_copy(
                ct_hbm.at[pl.ds(0, 1)], crows_v.at[pl.ds(0, 1)], sem_c).wait()
            return 0

        lax.fori_loop(0, CHR, drain, 0)

        pltpu.sync_copy(wrows_v, w_out.at[pl.ds(base + k * CHR, CHR)])
        pltpu.sync_copy(crows_v, c_out.at[pl.ds(base + k * CHR, CHR)])
        return 0

    lax.fori_loop(0, NCHK, chunk, 0)


def kernel(X, word_table, context_table):
    w = X[:, 0]
    c = X[:, 1]
    w_rows, c_rows = _dual_gather(w, c, word_table, context_table)
    return (w_rows[:, None, :], c_rows[:, None, :])
